# baseline (speedup 1.0000x reference)
import functools

import jax
import jax.numpy as jnp
from jax import lax
from jax.experimental import pallas as pl
from jax.experimental.pallas import tpu as pltpu
from jax.experimental.pallas import tpu_sc as plsc

A = 60
U = 64
R = A * U
NC = 2
NS = 16
NW = NC * NS
NRG = 8
ND = NW // NRG
RPW = R // NRG
NBLK = RPW // 16


def _sc_projector(volT, lin, w, L, Z):
    DS = Z // ND

    mesh = plsc.VectorSubcoreMesh(core_axis_name="c", subcore_axis_name="s")

    @functools.partial(
        pl.kernel,
        out_type=jax.ShapeDtypeStruct((ND, R, DS), jnp.float32),
        mesh=mesh,
        compiler_params=pltpu.CompilerParams(
            needs_layout_passes=False, use_tc_tiling_on_sc=False),
        scratch_types=[
            pltpu.VMEM((volT.shape[1],), jnp.float32),
            pltpu.VMEM((16, L), jnp.int32),
            pltpu.VMEM((16, L), jnp.float32),
            pltpu.VMEM((RPW, DS), jnp.float32),
        ],
    )
    def body(volT_hbm, lin_hbm, w_hbm, out_hbm, table_v, lin_v, w_v, out_v):
        wid = lax.axis_index("s") * NC + lax.axis_index("c")
        rg = wid // ND
        ds = wid % ND
        pltpu.sync_copy(volT_hbm.at[ds], table_v)
        lane = lax.iota(jnp.int32, 16)

        def block(b, carry):
            r0 = (rg * NBLK + b) * 16
            pltpu.sync_copy(lin_hbm.at[pl.ds(r0, 16)], lin_v)
            pltpu.sync_copy(w_hbm.at[pl.ds(r0, 16)], w_v)

            acc0 = tuple(jnp.zeros((16,), jnp.float32) for _ in range(DS))

            @plsc.parallel_loop(0, L, unroll=4, carry=acc0)
            def acc(l, acc):
                lcol = jnp.full((16,), 0, jnp.int32) + l
                raw = plsc.load_gather(lin_v, [lane, lcol])
                wv = plsc.load_gather(w_v, [lane, lcol])
                base = ((raw & 63) << 10) + ((raw >> 6) << 4)
                return tuple(
                    acc[d] + wv * plsc.load_gather(table_v, [base + d])
                    for d in range(DS)
                )
            rows = b * 16 + lane
            for d in range(DS):
                plsc.store_scatter(
                    out_v, [rows, jnp.full((16,), d, jnp.int32)], acc[d])
            return carry

        lax.fori_loop(0, NBLK, block, 0)
        pltpu.sync_copy(out_v, out_hbm.at[ds, pl.ds(rg * RPW, RPW)])

    return body(volT, lin, w)


def kernel(vol, seg_lin, seg_w, seg_ids):
    B, C, X, Y, Z = vol.shape
    L = seg_lin.size // R
    DS = Z // ND
    volT = vol.reshape(X * Y, ND, DS).transpose(1, 0, 2).reshape(ND, -1)
    out = _sc_projector(volT, seg_lin.reshape(R, L).astype(jnp.int32),
                        seg_w.reshape(R, L), L, Z)
    rays = out.reshape(ND, A, U, DS).transpose(2, 1, 0, 3)
    return rays.reshape(1, 1, U, A, Z)


# speedup vs baseline: 1.2589x; 1.2589x over previous
import functools

import jax
import jax.numpy as jnp
from jax import lax
from jax.experimental import pallas as pl
from jax.experimental.pallas import tpu as pltpu
from jax.experimental.pallas import tpu_sc as plsc

A = 60
U = 64
R = A * U
NC = 2
NS = 16
NW = NC * NS
NRG = 8
ND = NW // NRG
RPW = R // NRG
NBLK = RPW // 16
LPAD = 128
LC = 16
NCH = LPAD // LC


def _sc_projector(volT, linT, wT, Z):
    DS = Z // ND

    mesh = plsc.VectorSubcoreMesh(core_axis_name="c", subcore_axis_name="s")

    @functools.partial(
        pl.kernel,
        out_type=jax.ShapeDtypeStruct((ND, R, DS), jnp.float32),
        mesh=mesh,
        compiler_params=pltpu.CompilerParams(
            needs_layout_passes=False, use_tc_tiling_on_sc=False),
        scratch_types=[
            pltpu.VMEM((volT.shape[1],), jnp.float32),
            pltpu.VMEM((LC, RPW), jnp.int32),
            pltpu.VMEM((LC, RPW), jnp.int32),
            pltpu.VMEM((LC, RPW), jnp.float32),
            pltpu.VMEM((LC, RPW), jnp.float32),
            pltpu.VMEM((RPW, DS), jnp.float32),
            pltpu.SemaphoreType.DMA,
            pltpu.SemaphoreType.DMA,
            pltpu.SemaphoreType.DMA,
            pltpu.SemaphoreType.DMA,
        ],
    )
    def body(volT_hbm, linT_hbm, wT_hbm, out_hbm,
             table_v, lin_a, lin_b, w_a, w_b, out_v,
             sem_la, sem_lb, sem_wa, sem_wb):
        wid = lax.axis_index("s") * NC + lax.axis_index("c")
        rg = wid // ND
        ds = wid % ND
        pltpu.sync_copy(volT_hbm.at[ds], table_v)
        lane = lax.iota(jnp.int32, 16)

        lin_bufs = (lin_a, lin_b)
        w_bufs = (w_a, w_b)
        lin_sems = (sem_la, sem_lb)
        w_sems = (sem_wa, sem_wb)

        def issue(c):
            p = c % 2
            return (
                pltpu.async_copy(
                    linT_hbm.at[rg, pl.ds(c * LC, LC)], lin_bufs[p],
                    lin_sems[p]),
                pltpu.async_copy(
                    wT_hbm.at[rg, pl.ds(c * LC, LC)], w_bufs[p],
                    w_sems[p]),
            )

        pending = issue(0)
        for c in range(NCH):
            for h in pending:
                h.wait()
            if c + 1 < NCH:
                nxt = issue(c + 1)
            lin_v = lin_bufs[c % 2]
            w_v = w_bufs[c % 2]

            def block(b, carry):
                cols = pl.ds(b * 16, 16)
                acc = [jnp.zeros((16,), jnp.float32) for _ in range(DS)]
                for l in range(LC):
                    lv = lin_v[l, cols]
                    wv = w_v[l, cols]
                    for d in range(DS):
                        g = plsc.load_gather(table_v, [lv + d])
                        acc[d] = acc[d] + wv * g
                rows = b * 16 + lane
                for d in range(DS):
                    col = jnp.full((16,), d, jnp.int32)
                    if c == 0:
                        plsc.store_scatter(out_v, [rows, col], acc[d])
                    else:
                        plsc.addupdate_scatter(out_v, [rows, col], acc[d])
                return carry

            lax.fori_loop(0, NBLK, block, 0)
            if c + 1 < NCH:
                pending = nxt
        pltpu.sync_copy(out_v, out_hbm.at[ds, pl.ds(rg * RPW, RPW)])

    return body(volT, linT, wT)


def kernel(vol, seg_lin, seg_w, seg_ids):
    B, C, X, Y, Z = vol.shape
    L = seg_lin.size // R
    DS = Z // ND
    volT = vol.reshape(X * Y, ND, DS).transpose(1, 0, 2).reshape(ND, -1)
    lin = seg_lin.astype(jnp.int32)
    lin2 = ((lin & (X - 1)) << 10) | ((lin >> 6) << 4)
    linT = jnp.pad(lin2.reshape(R, L), ((0, 0), (0, LPAD - L))) \
              .T.reshape(LPAD, NRG, RPW).transpose(1, 0, 2)
    wT = jnp.pad(seg_w.reshape(R, L), ((0, 0), (0, LPAD - L))) \
            .T.reshape(LPAD, NRG, RPW).transpose(1, 0, 2)
    out = _sc_projector(volT, linT, wT, Z)
    rays = out.reshape(ND, A, U, DS).transpose(2, 1, 0, 3)
    return rays.reshape(1, 1, U, A, Z)


# speedup vs baseline: 1.9046x; 1.5129x over previous
import functools

import jax
import jax.numpy as jnp
from jax import lax
from jax.experimental import pallas as pl
from jax.experimental.pallas import tpu as pltpu
from jax.experimental.pallas import tpu_sc as plsc

A = 60
U = 64
R = A * U
NC = 2
NS = 16
NW = NC * NS
NRG = 8
ND = NW // NRG
RPW = R // NRG
NBLK = RPW // 16
LPAD = 128
LC = 16
NCH = LPAD // LC


def _sc_projector(volT, linT, wT, Z):
    DS = Z // ND

    mesh = plsc.VectorSubcoreMesh(core_axis_name="c", subcore_axis_name="s")

    @functools.partial(
        pl.kernel,
        out_type=jax.ShapeDtypeStruct((ND, R, DS), jnp.float32),
        mesh=mesh,
        compiler_params=pltpu.CompilerParams(
            needs_layout_passes=False, use_tc_tiling_on_sc=False),
        scratch_types=[
            pltpu.VMEM((volT.shape[1],), jnp.float32),
            pltpu.VMEM((LC, RPW), jnp.int32),
            pltpu.VMEM((LC, RPW), jnp.int32),
            pltpu.VMEM((LC, RPW), jnp.float32),
            pltpu.VMEM((LC, RPW), jnp.float32),
            pltpu.VMEM((RPW, DS), jnp.float32),
            pltpu.SemaphoreType.DMA,
            pltpu.SemaphoreType.DMA,
            pltpu.SemaphoreType.DMA,
            pltpu.SemaphoreType.DMA,
        ],
    )
    def body(volT_hbm, linT_hbm, wT_hbm, out_hbm,
             table_v, lin_a, lin_b, w_a, w_b, out_v,
             sem_la, sem_lb, sem_wa, sem_wb):
        wid = lax.axis_index("s") * NC + lax.axis_index("c")
        rg = wid // ND
        ds = wid % ND
        pltpu.sync_copy(volT_hbm.at[ds], table_v)
        lane = lax.iota(jnp.int32, 16)
        colk = [(lane + k) & 15 for k in range(DS)]

        lin_bufs = (lin_a, lin_b)
        w_bufs = (w_a, w_b)
        lin_sems = (sem_la, sem_lb)
        w_sems = (sem_wa, sem_wb)

        def issue(c):
            p = c % 2
            return (
                pltpu.async_copy(
                    linT_hbm.at[rg, pl.ds(c * LC, LC)], lin_bufs[p],
                    lin_sems[p]),
                pltpu.async_copy(
                    wT_hbm.at[rg, pl.ds(c * LC, LC)], w_bufs[p],
                    w_sems[p]),
            )

        pending = issue(0)
        for c in range(NCH):
            for h in pending:
                h.wait()
            if c + 1 < NCH:
                nxt = issue(c + 1)
            lin_v = lin_bufs[c % 2]
            w_v = w_bufs[c % 2]

            def block(b, carry):
                cols = pl.ds(b * 16, 16)
                acc = [jnp.zeros((16,), jnp.float32) for _ in range(DS)]
                for l in range(LC):
                    lv = lin_v[l, cols]
                    wv = w_v[l, cols]
                    for d in range(DS):
                        g = plsc.load_gather(table_v, [lv + colk[d]])
                        acc[d] = acc[d] + wv * g
                rows = b * 16 + lane
                for d in range(DS):
                    if c == 0:
                        plsc.store_scatter(out_v, [rows, colk[d]], acc[d])
                    else:
                        plsc.addupdate_scatter(out_v, [rows, colk[d]], acc[d])
                return carry

            lax.fori_loop(0, NBLK, block, 0)
            if c + 1 < NCH:
                pending = nxt
        pltpu.sync_copy(out_v, out_hbm.at[ds, pl.ds(rg * RPW, RPW)])

    return body(volT, linT, wT)


def kernel(vol, seg_lin, seg_w, seg_ids):
    B, C, X, Y, Z = vol.shape
    L = seg_lin.size // R
    DS = Z // ND
    volT = vol.reshape(X * Y, ND, DS).transpose(1, 0, 2).reshape(ND, -1)
    lin = seg_lin.astype(jnp.int32)
    lin2 = ((lin & (X - 1)) << 10) | ((lin >> 6) << 4)
    linT = jnp.pad(lin2.reshape(R, L), ((0, 0), (0, LPAD - L))) \
              .T.reshape(LPAD, NRG, RPW).transpose(1, 0, 2)
    wT = jnp.pad(seg_w.reshape(R, L), ((0, 0), (0, LPAD - L))) \
            .T.reshape(LPAD, NRG, RPW).transpose(1, 0, 2)
    out = _sc_projector(volT, linT, wT, Z)
    rays = out.reshape(ND, A, U, DS).transpose(2, 1, 0, 3)
    return rays.reshape(1, 1, U, A, Z)


# speedup vs baseline: 2.3157x; 1.2159x over previous
import functools

import jax
import jax.numpy as jnp
from jax import lax
from jax.experimental import pallas as pl
from jax.experimental.pallas import tpu as pltpu
from jax.experimental.pallas import tpu_sc as plsc

A = 60
U = 64
R = A * U
NC = 2
NS = 16
NW = NC * NS
NRG = 8
ND = NW // NRG
RPW = R // NRG
NBLK = RPW // 16
NBLKT = R // 16
LPAD = 128
LC = 16


def _sc_projector(volT, linB, wB, nchunks, Z):
    DS = Z // ND

    mesh = plsc.VectorSubcoreMesh(core_axis_name="c", subcore_axis_name="s")

    @functools.partial(
        pl.kernel,
        out_type=jax.ShapeDtypeStruct((ND, R, DS), jnp.float32),
        mesh=mesh,
        compiler_params=pltpu.CompilerParams(
            needs_layout_passes=False, use_tc_tiling_on_sc=False),
        scratch_types=[
            pltpu.VMEM((volT.shape[1],), jnp.float32),
            pltpu.VMEM((LPAD, 16), jnp.int32),
            pltpu.VMEM((LPAD, 16), jnp.int32),
            pltpu.VMEM((LPAD, 16), jnp.float32),
            pltpu.VMEM((LPAD, 16), jnp.float32),
            pltpu.VMEM((RPW, DS), jnp.float32),
            pltpu.VMEM((32,), jnp.int32),
            pltpu.SemaphoreType.DMA,
            pltpu.SemaphoreType.DMA,
            pltpu.SemaphoreType.DMA,
            pltpu.SemaphoreType.DMA,
        ],
    )
    def body(volT_hbm, linB_hbm, wB_hbm, nch_hbm, out_hbm,
             table_v, lin_a, lin_b, w_a, w_b, out_v, nch_v,
             sem_la, sem_lb, sem_wa, sem_wb):
        wid = lax.axis_index("s") * NC + lax.axis_index("c")
        rg = wid // ND
        ds = wid % ND
        pltpu.sync_copy(nch_hbm.at[rg], nch_v)
        pltpu.sync_copy(volT_hbm.at[ds], table_v)
        lane = lax.iota(jnp.int32, 16)
        zero16 = jnp.full((16,), 0, jnp.int32)
        colk = [(lane + k) & 15 for k in range(DS)]

        lin_bufs = (lin_a, lin_b)
        w_bufs = (w_a, w_b)
        lin_sems = (sem_la, sem_lb)
        w_sems = (sem_wa, sem_wb)

        def issue(b, p):
            gb = jnp.minimum(rg * NBLK + b, NBLKT - 1)
            return (
                pltpu.async_copy(linB_hbm.at[gb], lin_bufs[p], lin_sems[p]),
                pltpu.async_copy(wB_hbm.at[gb], w_bufs[p], w_sems[p]),
            )

        def wait(p):
            pltpu.make_async_copy(
                linB_hbm.at[0], lin_bufs[p], lin_sems[p]).wait()
            pltpu.make_async_copy(
                wB_hbm.at[0], w_bufs[p], w_sems[p]).wait()

        def compute(b, p):
            wait(p)
            lin_v = lin_bufs[p]
            w_v = w_bufs[p]
            nch_b = plsc.load_gather(nch_v, [zero16 + b])[0]

            def chunk(ci, acc):
                c16 = ci * LC
                for l in range(LC):
                    lv = lin_v[c16 + l]
                    wv = w_v[c16 + l]
                    acc = tuple(
                        acc[d] + wv * plsc.load_gather(
                            table_v, [lv + colk[d]])
                        for d in range(DS)
                    )
                return acc

            acc0 = tuple(jnp.zeros((16,), jnp.float32) for _ in range(DS))
            acc = lax.fori_loop(0, nch_b, chunk, acc0)
            rows = b * 16 + lane
            for d in range(DS):
                plsc.store_scatter(out_v, [rows, colk[d]], acc[d])

        issue(0, 0)

        def pair(i, carry):
            b0 = i * 2
            issue(b0 + 1, 1)
            compute(b0, 0)
            issue(b0 + 2, 0)
            compute(b0 + 1, 1)
            return carry

        lax.fori_loop(0, NBLK // 2, pair, 0)
        wait(0)
        pltpu.sync_copy(out_v, out_hbm.at[ds, pl.ds(rg * RPW, RPW)])

    return body(volT, linB, wB, nchunks)


def kernel(vol, seg_lin, seg_w, seg_ids):
    B, C, X, Y, Z = vol.shape
    L = seg_lin.size // R
    DS = Z // ND
    volT = vol.reshape(X * Y, ND, DS).transpose(1, 0, 2).reshape(ND, -1)
    lin = seg_lin.astype(jnp.int32)
    lin2 = ((lin & (X - 1)) << 10) | ((lin >> 6) << 4)
    linB = jnp.pad(lin2.reshape(R, L), ((0, 0), (0, LPAD - L))) \
              .reshape(NBLKT, 16, LPAD).transpose(0, 2, 1)
    wB = jnp.pad(seg_w.reshape(R, L), ((0, 0), (0, LPAD - L))) \
            .reshape(NBLKT, 16, LPAD).transpose(0, 2, 1)
    lastnz = jnp.max(
        jnp.where(seg_w.reshape(R, L) != 0.0,
                  jnp.arange(1, L + 1, dtype=jnp.int32), 0),
        axis=1)
    nchunks = (jnp.max(lastnz.reshape(NBLKT, 16), axis=1) + LC - 1) // LC
    nchunks = jnp.pad(nchunks.reshape(NRG, NBLK), ((0, 0), (0, 32 - NBLK)))
    out = _sc_projector(volT, linB, wB, nchunks, Z)
    rays = out.reshape(ND, A, U, DS).transpose(2, 1, 0, 3)
    return rays.reshape(1, 1, U, A, Z)


# speedup vs baseline: 3.1041x; 1.3405x over previous
import functools

import jax
import jax.numpy as jnp
from jax import lax
from jax.experimental import pallas as pl
from jax.experimental.pallas import tpu as pltpu
from jax.experimental.pallas import tpu_sc as plsc

A = 60
U = 64
R = A * U
NC = 2
NS = 16
NW = NC * NS
NRG = 8
ND = NW // NRG
RPW = R // NRG
NBLK = RPW // 16
NBLKT = R // 16
LPAD = 128
LC = 16


def _sc_projector(volT, linB, wB, nchunks, Z):
    DS = Z // ND

    mesh = plsc.VectorSubcoreMesh(core_axis_name="c", subcore_axis_name="s")

    @functools.partial(
        pl.kernel,
        out_type=jax.ShapeDtypeStruct((ND, R, DS), jnp.float32),
        mesh=mesh,
        compiler_params=pltpu.CompilerParams(
            needs_layout_passes=False, use_tc_tiling_on_sc=False),
        scratch_types=[
            pltpu.VMEM((volT.shape[1],), jnp.float32),
            pltpu.VMEM((16 * LPAD,), jnp.int32),
            pltpu.VMEM((16 * LPAD,), jnp.int32),
            pltpu.VMEM((16 * LPAD,), jnp.float32),
            pltpu.VMEM((16 * LPAD,), jnp.float32),
            pltpu.VMEM((RPW, DS), jnp.float32),
            pltpu.VMEM((32,), jnp.int32),
            pltpu.SemaphoreType.DMA,
            pltpu.SemaphoreType.DMA,
            pltpu.SemaphoreType.DMA,
            pltpu.SemaphoreType.DMA,
        ],
    )
    def body(volT_hbm, linB_hbm, wB_hbm, nch_hbm, out_hbm,
             table_v, lin_a, lin_b, w_a, w_b, out_v, nch_v,
             sem_la, sem_lb, sem_wa, sem_wb):
        wid = lax.axis_index("s") * NC + lax.axis_index("c")
        rg = wid // ND
        ds = wid % ND
        pltpu.sync_copy(nch_hbm.at[rg], nch_v)
        pltpu.sync_copy(volT_hbm.at[ds], table_v)
        lane = lax.iota(jnp.int32, 16)
        zero16 = jnp.full((16,), 0, jnp.int32)
        colk = [(lane + k) & 15 for k in range(DS)]
        rawk = [lane * LPAD + colk[k] for k in range(LC)]

        lin_bufs = (lin_a, lin_b)
        w_bufs = (w_a, w_b)
        lin_sems = (sem_la, sem_lb)
        w_sems = (sem_wa, sem_wb)

        def issue(b, p):
            gb = jnp.minimum(rg * NBLK + b, NBLKT - 1)
            return (
                pltpu.async_copy(linB_hbm.at[gb], lin_bufs[p], lin_sems[p]),
                pltpu.async_copy(wB_hbm.at[gb], w_bufs[p], w_sems[p]),
            )

        def wait(p):
            pltpu.make_async_copy(
                linB_hbm.at[0], lin_bufs[p], lin_sems[p]).wait()
            pltpu.make_async_copy(
                wB_hbm.at[0], w_bufs[p], w_sems[p]).wait()

        def compute(b, p):
            wait(p)
            lin_v = lin_bufs[p]
            w_v = w_bufs[p]
            nch_b = plsc.load_gather(nch_v, [zero16 + b])[0]

            def chunk(ci, acc):
                c16 = ci * LC
                for k in range(LC):
                    lv = plsc.load_gather(lin_v, [c16 + rawk[k]])
                    wv = plsc.load_gather(w_v, [c16 + rawk[k]])
                    acc = tuple(
                        acc[d] + wv * plsc.load_gather(
                            table_v, [lv + colk[d]])
                        for d in range(DS)
                    )
                return acc

            acc0 = tuple(jnp.zeros((16,), jnp.float32) for _ in range(DS))
            acc = lax.fori_loop(0, nch_b, chunk, acc0)
            rows = b * 16 + lane
            for d in range(DS):
                plsc.store_scatter(out_v, [rows, colk[d]], acc[d])

        issue(0, 0)

        def pair(i, carry):
            b0 = i * 2
            issue(b0 + 1, 1)
            compute(b0, 0)
            issue(b0 + 2, 0)
            compute(b0 + 1, 1)
            return carry

        lax.fori_loop(0, NBLK // 2, pair, 0)
        wait(0)
        pltpu.sync_copy(out_v, out_hbm.at[ds, pl.ds(rg * RPW, RPW)])

    return body(volT, linB, wB, nchunks)


def kernel(vol, seg_lin, seg_w, seg_ids):
    B, C, X, Y, Z = vol.shape
    L = seg_lin.size // R
    DS = Z // ND
    volT = vol.reshape(X * Y, ND, DS).transpose(1, 0, 2).reshape(ND, -1)
    lin = seg_lin.astype(jnp.int32)
    lin2 = ((lin & (X - 1)) << 10) | ((lin >> 6) << 4)
    linB = jnp.pad(lin2.reshape(R, L), ((0, 0), (0, LPAD - L))) \
              .reshape(NBLKT, 16 * LPAD)
    wB = jnp.pad(seg_w.reshape(R, L), ((0, 0), (0, LPAD - L))) \
            .reshape(NBLKT, 16 * LPAD)
    lastnz = jnp.max(
        jnp.where(seg_w.reshape(R, L) != 0.0,
                  jnp.arange(1, L + 1, dtype=jnp.int32), 0),
        axis=1)
    nchunks = (jnp.max(lastnz.reshape(NBLKT, 16), axis=1) + LC - 1) // LC
    nchunks = jnp.pad(nchunks.reshape(NRG, NBLK), ((0, 0), (0, 32 - NBLK)))
    out = _sc_projector(volT, linB, wB, nchunks, Z)
    rays = out.reshape(ND, A, U, DS).transpose(2, 1, 0, 3)
    return rays.reshape(1, 1, U, A, Z)


# speedup vs baseline: 3.1581x; 1.0174x over previous
import functools

import jax
import jax.numpy as jnp
from jax import lax
from jax.experimental import pallas as pl
from jax.experimental.pallas import tpu as pltpu
from jax.experimental.pallas import tpu_sc as plsc

A = 60
U = 64
R = A * U
NC = 2
NS = 16
NW = NC * NS
NRG = 8
ND = NW // NRG
RPW = R // NRG
NBLK = RPW // 16
NBLKT = R // 16
LPAD = 128
LC = 16


def _sc_projector(volT, linB, wB, nchunks, Z):
    DS = Z // ND

    mesh = plsc.VectorSubcoreMesh(core_axis_name="c", subcore_axis_name="s")

    @functools.partial(
        pl.kernel,
        out_type=jax.ShapeDtypeStruct((U, A, Z), jnp.float32),
        mesh=mesh,
        compiler_params=pltpu.CompilerParams(
            needs_layout_passes=False, use_tc_tiling_on_sc=False),
        scratch_types=[
            pltpu.VMEM((volT.shape[1],), jnp.float32),
            pltpu.VMEM((16 * LPAD,), jnp.int32),
            pltpu.VMEM((16 * LPAD,), jnp.int32),
            pltpu.VMEM((16 * LPAD,), jnp.float32),
            pltpu.VMEM((16 * LPAD,), jnp.float32),
            pltpu.VMEM((RPW, DS), jnp.float32),
            pltpu.VMEM((32,), jnp.int32),
            pltpu.SemaphoreType.DMA,
            pltpu.SemaphoreType.DMA,
            pltpu.SemaphoreType.DMA,
            pltpu.SemaphoreType.DMA,
            pltpu.SemaphoreType.DMA,
        ],
    )
    def body(volT_hbm, linB_hbm, wB_hbm, nch_hbm, out_hbm,
             table_v, lin_a, lin_b, w_a, w_b, out_v, nch_v,
             sem_la, sem_lb, sem_wa, sem_wb, sem_out):
        wid = lax.axis_index("s") * NC + lax.axis_index("c")
        rg = wid // ND
        ds = wid % ND
        pltpu.sync_copy(nch_hbm.at[rg], nch_v)
        pltpu.sync_copy(volT_hbm.at[ds], table_v)
        lane = lax.iota(jnp.int32, 16)
        zero16 = jnp.full((16,), 0, jnp.int32)
        colk = [(lane + k) & 15 for k in range(DS)]
        rawk = [lane * LPAD + colk[k] for k in range(LC)]

        lin_bufs = (lin_a, lin_b)
        w_bufs = (w_a, w_b)
        lin_sems = (sem_la, sem_lb)
        w_sems = (sem_wa, sem_wb)

        def issue(b, p):
            gb = jnp.minimum(rg * NBLK + b, NBLKT - 1)
            return (
                pltpu.async_copy(linB_hbm.at[gb], lin_bufs[p], lin_sems[p]),
                pltpu.async_copy(wB_hbm.at[gb], w_bufs[p], w_sems[p]),
            )

        def wait(p):
            pltpu.make_async_copy(
                linB_hbm.at[0], lin_bufs[p], lin_sems[p]).wait()
            pltpu.make_async_copy(
                wB_hbm.at[0], w_bufs[p], w_sems[p]).wait()

        def compute(b, p):
            wait(p)
            lin_v = lin_bufs[p]
            w_v = w_bufs[p]
            nch_b = plsc.load_gather(nch_v, [zero16 + b])[0]

            def chunk(ci, acc):
                c16 = ci * LC
                for k in range(LC):
                    lv = plsc.load_gather(lin_v, [c16 + rawk[k]])
                    wv = plsc.load_gather(w_v, [c16 + rawk[k]])
                    acc = tuple(
                        acc[d] + wv * plsc.load_gather(
                            table_v, [lv + colk[d]])
                        for d in range(DS)
                    )
                return acc

            acc0 = tuple(jnp.zeros((16,), jnp.float32) for _ in range(DS))
            acc = lax.fori_loop(0, nch_b, chunk, acc0)
            rows = b * 16 + lane
            for d in range(DS):
                plsc.store_scatter(out_v, [rows, colk[d]], acc[d])
            r0 = (rg * NBLK + b) * 16
            pltpu.async_copy(
                out_v.at[pl.ds(b * 16, 16)],
                out_hbm.at[pl.ds(r0 % U, 16), r0 // U,
                           pl.ds(ds * DS, DS)],
                sem_out)

        issue(0, 0)

        def pair(i, carry):
            b0 = i * 2
            issue(b0 + 1, 1)
            compute(b0, 0)
            issue(b0 + 2, 0)
            compute(b0 + 1, 1)
            return carry

        lax.fori_loop(0, NBLK // 2, pair, 0)
        wait(0)
        for _ in range(NBLK):
            pltpu.make_async_copy(
                out_v.at[pl.ds(0, 16)],
                out_hbm.at[pl.ds(0, 16), 0, pl.ds(0, DS)],
                sem_out).wait()

    return body(volT, linB, wB, nchunks)


def kernel(vol, seg_lin, seg_w, seg_ids):
    B, C, X, Y, Z = vol.shape
    L = seg_lin.size // R
    DS = Z // ND
    volT = vol.reshape(X * Y, ND, DS).transpose(1, 0, 2).reshape(ND, -1)
    lin = seg_lin.astype(jnp.int32)
    lin2 = ((lin & (X - 1)) << 10) | ((lin >> 6) << 4)
    linB = jnp.pad(lin2.reshape(R, L), ((0, 0), (0, LPAD - L))) \
              .reshape(NBLKT, 16 * LPAD)
    wB = jnp.pad(seg_w.reshape(R, L), ((0, 0), (0, LPAD - L))) \
            .reshape(NBLKT, 16 * LPAD)
    lastnz = jnp.max(
        jnp.where(seg_w.reshape(R, L) != 0.0,
                  jnp.arange(1, L + 1, dtype=jnp.int32), 0),
        axis=1)
    nchunks = (jnp.max(lastnz.reshape(NBLKT, 16), axis=1) + LC - 1) // LC
    nchunks = jnp.pad(nchunks.reshape(NRG, NBLK), ((0, 0), (0, 32 - NBLK)))
    out = _sc_projector(volT, linB, wB, nchunks, Z)
    return out.reshape(1, 1, U, A, Z)
